# Initial kernel scaffold; baseline (speedup 1.0000x reference)
import jax
import jax.numpy as jnp
from jax.experimental import pallas as pl


def kernel(coords):
    raise NotImplementedError("write your pallas kernel here")


# speedup vs baseline: 5.9751x; 5.9751x over previous
import functools

import jax
import jax.numpy as jnp
from jax.experimental import pallas as pl
from jax.experimental.pallas import tpu as pltpu

NUM_RBF = 32
MIN_D = 2.0
MAX_D = 22.0
K = 30
NPAD = 5120
ROWS = 256
PAD_COORD = 1.0e9


def _knn_rbf_kernel(rows_ref, colsT_ref, idx_ref, rbf_ref):
    xr = rows_ref[...]
    ct = colsT_ref[...]
    d2 = ((xr[:, 0:1] - ct[0:1, :]) ** 2
          + (xr[:, 1:2] - ct[1:2, :]) ** 2
          + (xr[:, 2:3] - ct[2:3, :]) ** 2)

    iota = jax.lax.broadcasted_iota(jnp.int32, d2.shape, 1)
    big_idx = jnp.int32(NPAD)
    step = (MAX_D - MIN_D) / (NUM_RBF - 1)
    mu = MIN_D + jax.lax.broadcasted_iota(
        jnp.int32, (1, NUM_RBF), 1).astype(jnp.float32) * step
    sigma = (MAX_D - MIN_D) / NUM_RBF
    inv2s2 = 1.0 / (2.0 * sigma * sigma)

    idx_ref[...] = jnp.zeros((ROWS, 32), jnp.int32)
    work = d2
    for k in range(K + 1):
        m = jnp.min(work, axis=1, keepdims=True)
        am = jnp.min(jnp.where(work == m, iota, big_idx),
                     axis=1, keepdims=True)
        work = jnp.where(iota == am, jnp.inf, work)
        if k == 0:
            continue
        dist = jnp.sqrt(jnp.maximum(m, 1e-12))
        mask = dist <= MAX_D
        dist = jnp.where(mask, dist, 0.0)
        rbf_k = jnp.exp(-((dist - mu) ** 2) * inv2s2)
        rbf_k = rbf_k * mask.astype(rbf_k.dtype)
        idx_ref[:, k - 1:k] = am
        rbf_ref[:, k - 1, :] = rbf_k


@jax.jit
def kernel(coords):
    n = coords.shape[0]
    cpad = jnp.concatenate(
        [coords, jnp.full((NPAD - n, 3), PAD_COORD, coords.dtype)], axis=0)
    rows = jnp.zeros((NPAD, 128), jnp.float32).at[:, :3].set(cpad)
    colsT = jnp.zeros((8, NPAD), jnp.float32).at[:3, :].set(cpad.T)

    idx_out, rbf = pl.pallas_call(
        _knn_rbf_kernel,
        grid=(NPAD // ROWS,),
        in_specs=[
            pl.BlockSpec((ROWS, 128), lambda i: (i, 0)),
            pl.BlockSpec((8, NPAD), lambda i: (0, 0)),
        ],
        out_specs=[
            pl.BlockSpec((ROWS, 32), lambda i: (i, 0)),
            pl.BlockSpec((ROWS, K, NUM_RBF), lambda i: (i, 0, 0)),
        ],
        out_shape=[
            jax.ShapeDtypeStruct((NPAD, 32), jnp.int32),
            jax.ShapeDtypeStruct((NPAD, K, NUM_RBF), jnp.float32),
        ],
        compiler_params=pltpu.CompilerParams(
            dimension_semantics=("parallel",),
        ),
    )(rows, colsT)

    src = jnp.repeat(jnp.arange(n, dtype=jnp.int32), K)
    dst = idx_out[:n, :K].reshape(-1)
    edge_index = jnp.stack([src, dst], axis=0)
    return edge_index, rbf[:n].reshape(n * K, NUM_RBF)
